# baseline (speedup 1.0000x reference)
import functools

import jax
import jax.numpy as jnp
from jax import lax
from jax.experimental import pallas as pl
from jax.experimental.pallas import tpu as pltpu
from jax.experimental.pallas import tpu_sc as plsc

N_BITS = 2
N_LEVELS = 2 ** N_BITS

NC = 2
NS = 16
NW = NC * NS
LANES = 16
CHUNK = 16384
N_ACC = 13


def _vec_body(buf, i, acc, t1, t2, t3, av, tn, ones, zeros):
    (s1, s2, s3, S, P1, P2, P3, Q, R1, R2, R3, D, E) = acc
    x = buf[pl.ds(i * LANES, LANES)]
    m1 = x > t1
    m2 = x >= t2
    m3 = x > t3
    c1 = jnp.where(m1, ones, zeros)
    c2 = jnp.where(m2, ones, zeros)
    c3 = jnp.where(m3, ones, zeros)
    y1 = jnp.where(m1, x, zeros)
    y2 = jnp.where(m2, x, zeros)
    y3 = jnp.where(m3, x, zeros)
    s1 = s1 + c1
    s2 = s2 + c2
    s3 = s3 + c3
    P1 = P1 + y1
    P2 = P2 + y2
    P3 = P3 + y3
    S = S + x
    Q = Q + x * x
    R1 = R1 + y1 * x
    R2 = R2 + y2 * x
    R3 = R3 + y3 * x
    b = c1 + c2 + c3
    d = x - av * b
    ad = jnp.abs(d)
    D = D + ad
    E = E + jnp.where(ad < tn, ones, zeros)
    return (s1, s2, s3, S, P1, P2, P3, Q, R1, R2, R3, D, E)


@functools.partial(jax.jit, static_argnames=("n",))
def _sc_reduce(acts_flat, alpha_vec, *, n):
    per_w = n // NW
    n_chunks = per_w // CHUNK
    assert per_w * NW == n and n_chunks * CHUNK == per_w and n_chunks % 2 == 0

    mesh = plsc.VectorSubcoreMesh(core_axis_name="c", subcore_axis_name="s",
                                  num_cores=NC, num_subcores=NS)

    @functools.partial(
        pl.kernel,
        out_type=jax.ShapeDtypeStruct((NW, N_ACC * LANES), jnp.float32),
        mesh=mesh,
        scratch_types=[
            pltpu.VMEM((CHUNK,), jnp.float32),
            pltpu.VMEM((CHUNK,), jnp.float32),
            pltpu.VMEM((LANES,), jnp.float32),
            pltpu.VMEM((N_ACC * LANES,), jnp.float32),
            pltpu.SemaphoreType.DMA,
            pltpu.SemaphoreType.DMA,
        ],
    )
    def sc_kernel(acts_hbm, av_hbm, out_hbm, buf0, buf1, pv, stage, sem0, sem1):
        cid = lax.axis_index("c")
        sid = lax.axis_index("s")
        wid = sid * NC + cid
        base = wid * per_w

        pltpu.sync_copy(av_hbm, pv)
        av = pv[...]
        t1 = av * 0.5
        t2 = av * 1.5
        t3 = av * 2.5
        tn = av * 0.01
        ones = jnp.full((LANES,), 1.0, jnp.float32)
        zeros = jnp.full((LANES,), 0.0, jnp.float32)

        bufs = (buf0, buf1)
        sems = (sem0, sem1)

        def issue(c, p):
            pltpu.async_copy(acts_hbm.at[pl.ds(base + c * CHUNK, CHUNK)],
                             bufs[p], sems[p])

        def wait(p):
            pltpu.make_async_copy(acts_hbm.at[pl.ds(base, CHUNK)],
                                  bufs[p], sems[p]).wait()

        def process(p, acc):
            return lax.fori_loop(
                0, CHUNK // LANES,
                lambda i, a: _vec_body(bufs[p], i, a, t1, t2, t3, av, tn,
                                       ones, zeros),
                acc, unroll=2)

        issue(0, 0)

        acc0 = (jnp.zeros((LANES,), jnp.float32),) * N_ACC

        def pair_body(g, acc):
            c = 2 * g
            issue(c + 1, 1)
            wait(0)
            acc = process(0, acc)
            issue(c + 2, 0)
            wait(1)
            acc = process(1, acc)
            return acc

        acc = lax.fori_loop(0, n_chunks // 2 - 1, pair_body, acc0)
        issue(n_chunks - 1, 1)
        wait(0)
        acc = process(0, acc)
        wait(1)
        acc = process(1, acc)

        for r in range(N_ACC):
            stage[pl.ds(r * LANES, LANES)] = acc[r]
        pltpu.sync_copy(stage, out_hbm.at[wid])

    return sc_kernel(acts_flat, alpha_vec)


def kernel(activations, alpha):
    n = activations.size
    dt = jnp.float32
    acts_flat = activations.reshape(-1)
    alpha = alpha.astype(dt)
    alpha_vec = jnp.full((LANES,), alpha, dt)

    parts = _sc_reduce(acts_flat, alpha_vec, n=n)
    tot = parts.reshape(NW, N_ACC, LANES).sum(axis=(0, 2))
    s1, s2, s3, S, P1, P2, P3, Q, R1, R2, R3, D, E = [tot[i] for i in range(N_ACC)]

    nf = jnp.asarray(n, dt)
    cnt = jnp.stack([nf - s1, s1 - s2, s2 - s3, s3])
    bsum = jnp.stack([S - P1, P1 - P2, P2 - P3, P3])
    bsq = jnp.stack([Q - R1, R1 - R2, R2 - R3, R3])

    levels = jnp.arange(N_LEVELS, dtype=dt) * alpha
    safe = jnp.maximum(cnt, 1.0)
    mean = bsum / safe
    mse = jnp.where(cnt > 0, (mean - levels) ** 2, 0.0)
    total_mse = jnp.sum(mse)
    var = (bsq - 2.0 * mean * bsum + cnt * mean * mean) / safe
    total_var = jnp.sum(jnp.where(cnt >= 2, var, 0.0))
    loss = total_mse + total_var

    qsq = bsq - 2.0 * levels * bsum + cnt * levels * levels
    quantization_mse = jnp.sum(qsq) / nf
    mean_distance = D / nf
    max_dist = alpha * 0.5
    effectiveness = jnp.clip(100.0 * (1.0 - mean_distance / (max_dist + 1e-12)),
                             0.0, 100.0)
    near_levels = (E / nf) * 100.0
    return (loss, total_mse, total_var, quantization_mse, mean_distance,
            effectiveness, near_levels)
